# Initial kernel scaffold; baseline (speedup 1.0000x reference)
import jax
import jax.numpy as jnp
from jax.experimental import pallas as pl


def kernel(target, context, neg_samples, in_embed, out_embed):
    raise NotImplementedError("write your pallas kernel here")


# speedup vs baseline: 1.5937x; 1.5937x over previous
import functools

import jax
import jax.numpy as jnp
from jax import lax
from jax.experimental import pallas as pl
from jax.experimental.pallas import tpu as pltpu
from jax.experimental.pallas import tpu_sc as plsc

B = 16384
K = 5
D = 64
NC = 2
NS = 16
L = 16
NW = NC * NS
BPW = B // NW
CHUNK = 256
NCHUNK = BPW // CHUNK
GATHER_ROWS = 128


def _sc_body(target_hbm, context_hbm, neg_hbm, in_embed_hbm, out_embed_hbm,
             pos_hbm, neg_out_hbm,
             idx_t, idx_c, idx_n, tgt_v, ctx_v, neg_v, pos_v, negsc_v, sem):
    wid = lax.axis_index("s") * NC + lax.axis_index("c")
    base = wid * BPW
    lane = lax.iota(jnp.int32, L)

    for ci in range(NCHUNK):
        cbase = base + ci * CHUNK
        pltpu.sync_copy(target_hbm.at[pl.ds(cbase, CHUNK)], idx_t)
        pltpu.sync_copy(context_hbm.at[pl.ds(cbase, CHUNK)], idx_c)
        pltpu.sync_copy(neg_hbm.at[pl.ds(cbase * K, CHUNK * K)], idx_n)

        copies = []
        for j in range(CHUNK // GATHER_ROWS):
            s = pl.ds(j * GATHER_ROWS, GATHER_ROWS)
            copies.append(pltpu.async_copy(
                in_embed_hbm.at[idx_t.at[s]], tgt_v.at[s], sem))
            copies.append(pltpu.async_copy(
                out_embed_hbm.at[idx_c.at[s]], ctx_v.at[s], sem))
        for j in range(CHUNK * K // GATHER_ROWS):
            s = pl.ds(j * GATHER_ROWS, GATHER_ROWS)
            copies.append(pltpu.async_copy(
                out_embed_hbm.at[idx_n.at[s]], neg_v.at[s], sem))
        for cp in copies:
            cp.wait()

        def group_body(g, _):
            row_tc = g * L + lane
            rows_n = [row_tc * K + k for k in range(K)]

            def d_body(d, accs):
                acc_p = accs[0]
                col = jnp.full((L,), d, jnp.int32)
                t = plsc.load_gather(tgt_v, [row_tc, col])
                c = plsc.load_gather(ctx_v, [row_tc, col])
                acc_p = acc_p + t * c
                new_accs = [acc_p]
                for k in range(K):
                    n = plsc.load_gather(neg_v, [rows_n[k], col])
                    new_accs.append(accs[k + 1] + t * n)
                return tuple(new_accs)

            zeros = jnp.zeros((L,), jnp.float32)
            accs = lax.fori_loop(0, D, d_body, (zeros,) * (K + 1))

            off = ci * CHUNK + g * L
            plsc.store_scatter(pos_v, [off + lane], accs[0])
            for k in range(K):
                plsc.store_scatter(negsc_v, [(off + lane) * K + k],
                                   accs[k + 1])
            return 0

        lax.fori_loop(0, CHUNK // L, group_body, 0)

    pltpu.sync_copy(pos_v, pos_hbm.at[pl.ds(base, BPW)])
    pltpu.sync_copy(negsc_v, neg_out_hbm.at[pl.ds(base * K, BPW * K)])


_sc_scores = pl.kernel(
    _sc_body,
    out_type=(jax.ShapeDtypeStruct((B,), jnp.float32),
              jax.ShapeDtypeStruct((B * K,), jnp.float32)),
    mesh=plsc.VectorSubcoreMesh(core_axis_name="c", subcore_axis_name="s"),
    scratch_types=(
        pltpu.VMEM((CHUNK,), jnp.int32),
        pltpu.VMEM((CHUNK,), jnp.int32),
        pltpu.VMEM((CHUNK * K,), jnp.int32),
        pltpu.VMEM((CHUNK, D), jnp.float32),
        pltpu.VMEM((CHUNK, D), jnp.float32),
        pltpu.VMEM((CHUNK * K, D), jnp.float32),
        pltpu.VMEM((BPW,), jnp.float32),
        pltpu.VMEM((BPW * K,), jnp.float32),
        pltpu.SemaphoreType.DMA,
    ),
    compiler_params=pltpu.CompilerParams(needs_layout_passes=False,
                                         use_tc_tiling_on_sc=False),
)


def _loss_body(pos_ref, neg_ref, out_ref):
    lp = jnp.sum(jnp.log(jax.nn.sigmoid(pos_ref[...])))
    ln = jnp.sum(jnp.log(jax.nn.sigmoid(-neg_ref[...])))
    out_ref[0, 0] = -(lp / B + ln / (B * K))


_loss_kernel = pl.pallas_call(
    _loss_body,
    out_shape=jax.ShapeDtypeStruct((1, 1), jnp.float32),
    out_specs=pl.BlockSpec(memory_space=pltpu.SMEM),
)


@jax.jit
def kernel(target, context, neg_samples, in_embed, out_embed):
    pos_score, neg_score = _sc_scores(
        target.astype(jnp.int32), context.astype(jnp.int32),
        neg_samples.astype(jnp.int32), in_embed, out_embed)
    loss = _loss_kernel(pos_score.reshape(B // 128, 128),
                        neg_score.reshape(B * K // 128, 128))
    return loss[0, 0]
